# Initial kernel scaffold; baseline (speedup 1.0000x reference)
import jax
import jax.numpy as jnp
from jax.experimental import pallas as pl


def kernel(node_features, edge_features, flow_features, edge_index, params):
    raise NotImplementedError("write your pallas kernel here")


# speedup vs baseline: 3.5463x; 3.5463x over previous
import jax
import jax.numpy as jnp
from jax import lax
from jax.experimental import pallas as pl
from jax.experimental.pallas import tpu as pltpu
from jax.experimental.pallas import tpu_sc as plsc

N = 10000
E = 320000
D = 128
DF = 16
P = 2

NC = 2
NS = 16
NW = NC * NS
EPW = E // NW
CCH = 400
KIDX = 80
KC = CCH // KIDX
NCHUNK = EPW // CCH
NPAD = 10240
SLAB = NPAD // NS
CCH_S = 320
KC_S = CCH_S // KIDX
NCH_S = E // CCH_S

BE = 512
GE = E // BE
BN = 1000
GN = N // BN

_F32 = jnp.float32


def _ln_silu(h, g, b):
    m = jnp.mean(h, axis=-1, keepdims=True)
    c = h - m
    v = jnp.mean(c * c, axis=-1, keepdims=True)
    hn = c * lax.rsqrt(v + 1e-5) * g + b
    return hn * jax.nn.sigmoid(hn)


def _dot(a, b):
    return jnp.dot(a, b, preferred_element_type=_F32)


def _tables_body(n_ref, f_ref, ws_ref, wd_ref, wfs_ref, wfd_ref, b1_ref,
                 gs_ref, gd_ref):
    n = n_ref[...]
    f = f_ref[...]
    gs_ref[...] = _dot(n, ws_ref[...]) + _dot(f, wfs_ref[...]) + b1_ref[...]
    gd_ref[...] = _dot(n, wd_ref[...]) + _dot(f, wfd_ref[...])


def _tables(nfeat, fpad, ws, wd, wfs, wfd, b1):
    row = pl.BlockSpec((BN, D), lambda i: (i, 0))
    wsp = pl.BlockSpec((D, D), lambda i: (0, 0))
    vsp = pl.BlockSpec((1, D), lambda i: (0, 0))
    return pl.pallas_call(
        _tables_body,
        grid=(GN,),
        in_specs=[row, row, wsp, wsp, wsp, wsp, vsp],
        out_specs=[row, row],
        out_shape=[jax.ShapeDtypeStruct((N, D), _F32),
                   jax.ShapeDtypeStruct((N, D), _F32)],
    )(nfeat, fpad, ws, wd, wfs, wfd, b1)


def _edge_body(e_ref, x_ref, w1_ref, w2_ref, lg_ref, lb_ref, b2_ref, o_ref):
    e = e_ref[...]
    h = _dot(e, w1_ref[...]) + x_ref[...]
    a = _ln_silu(h, lg_ref[...], lb_ref[...])
    o_ref[...] = _dot(a, w2_ref[...]) + b2_ref[...] + e


def _edge_mlp(efeat, x, w1e, w2, lg, lb, b2):
    row = pl.BlockSpec((BE, D), lambda i: (i, 0))
    wsp = pl.BlockSpec((D, D), lambda i: (0, 0))
    vsp = pl.BlockSpec((1, D), lambda i: (0, 0))
    return pl.pallas_call(
        _edge_body,
        grid=(GE,),
        in_specs=[row, row, wsp, wsp, vsp, vsp, vsp],
        out_specs=row,
        out_shape=jax.ShapeDtypeStruct((E, D), _F32),
    )(efeat, x, w1e, w2, lg, lb, b2)


def _node_body(n_ref, p_ref, f_ref, wn_ref, wa_ref, wf_ref, b1_ref,
               lg_ref, lb_ref, w2_ref, b2_ref, o_ref):
    n = n_ref[...]
    agg = p_ref[0] + p_ref[1]
    h = _dot(n, wn_ref[...]) + _dot(agg, wa_ref[...]) + _dot(f_ref[...], wf_ref[...]) + b1_ref[...]
    a = _ln_silu(h, lg_ref[...], lb_ref[...])
    o_ref[...] = _dot(a, w2_ref[...]) + b2_ref[...] + n


def _node_mlp(nfeat, p, fpad, wn, wa, wf, b1, lg, lb, w2, b2):
    row = pl.BlockSpec((BN, D), lambda i: (i, 0))
    psp = pl.BlockSpec((2, BN, D), lambda i: (0, i, 0))
    wsp = pl.BlockSpec((D, D), lambda i: (0, 0))
    vsp = pl.BlockSpec((1, D), lambda i: (0, 0))
    return pl.pallas_call(
        _node_body,
        grid=(GN,),
        in_specs=[row, psp, row, wsp, wsp, wsp, vsp, vsp, vsp, wsp, vsp],
        out_specs=row,
        out_shape=jax.ShapeDtypeStruct((N, D), _F32),
    )(nfeat, p, fpad, wn, wa, wf, b1, lg, lb, w2, b2)


def _sc_mesh():
    return plsc.VectorSubcoreMesh(
        core_axis_name="c", subcore_axis_name="s", num_cores=NC, num_subcores=NS)


def _sc_gather_body(gs_hbm, gd_hbm, src_hbm, dst_hbm, out_hbm,
                    idx_s, idx_d, rows, sem):
    wid = lax.axis_index("s") * NC + lax.axis_index("c")
    base_w = wid * EPW

    def chunk(ci, carry):
        base = base_w + ci * CCH
        pltpu.sync_copy(src_hbm.at[pl.ds(base, CCH)], idx_s)
        pltpu.sync_copy(dst_hbm.at[pl.ds(base, CCH)], idx_d)
        cps = [pltpu.async_copy(gs_hbm.at[idx_s.at[pl.ds(k * KIDX, KIDX)]],
                                rows.at[pl.ds(k * KIDX, KIDX)], sem)
               for k in range(KC)]
        for cp in cps:
            cp.wait()
        cps = [pltpu.async_copy(gd_hbm.at[idx_d.at[pl.ds(k * KIDX, KIDX)]],
                                rows.at[pl.ds(k * KIDX, KIDX)], sem, add=True)
               for k in range(KC)]
        for cp in cps:
            cp.wait()
        pltpu.sync_copy(rows, out_hbm.at[pl.ds(base, CCH)])
        return carry

    lax.fori_loop(0, NCHUNK, chunk, 0)


def _sc_gather(gs, gd, src, dst):
    return pl.kernel(
        _sc_gather_body,
        out_type=jax.ShapeDtypeStruct((E, D), _F32),
        mesh=_sc_mesh(),
        scratch_types=[
            pltpu.VMEM((CCH,), jnp.int32),
            pltpu.VMEM((CCH,), jnp.int32),
            pltpu.VMEM((CCH, D), _F32),
            pltpu.SemaphoreType.DMA,
        ],
    )(gs, gd, src, dst)


def _sc_scatter_body(e_hbm, dst_hbm, out_hbm, idx2, rows, acc, sem):
    cid = lax.axis_index("c")
    sid = lax.axis_index("s")
    wid = sid * NC + cid
    zero = jnp.zeros((16,), _F32)

    def zrow(i, carry):
        for j in range(D // 16):
            rows[i, pl.ds(j * 16, 16)] = zero
        return carry

    lax.fori_loop(0, CCH_S, zrow, 0)
    z0 = sid * SLAB
    pltpu.sync_copy(rows, acc.at[pl.ds(z0, CCH_S)])
    pltpu.sync_copy(rows, acc.at[pl.ds(z0 + CCH_S, CCH_S)])
    plsc.subcore_barrier()

    def chunk(j, carry):
        base = (wid + j * NW) * CCH_S
        for k in range(KC_S):
            pltpu.sync_copy(dst_hbm.at[pl.ds(base + k * KIDX, KIDX)], idx2.at[k])
        pltpu.async_copy(e_hbm.at[pl.ds(base, CCH_S)], rows, sem).wait()
        for k in range(KC_S):
            pltpu.sync_copy(rows.at[pl.ds(k * KIDX, KIDX)],
                            acc.at[idx2.at[k]], add=True)
        return carry

    nj = (NCH_S - 1 - wid) // NW + 1
    lax.fori_loop(0, nj, chunk, 0)
    plsc.subcore_barrier()
    pltpu.sync_copy(acc.at[pl.ds(sid * SLAB, SLAB)],
                    out_hbm.at[pl.ds(cid * NPAD + sid * SLAB, SLAB)])


def _sc_scatter(efeat, dst):
    return pl.kernel(
        _sc_scatter_body,
        out_type=jax.ShapeDtypeStruct((2 * NPAD, D), _F32),
        mesh=_sc_mesh(),
        scratch_types=[
            pltpu.VMEM((KC_S, KIDX), jnp.int32),
            pltpu.VMEM((CCH_S, D), _F32),
            pltpu.VMEM_SHARED((NPAD, D), _F32),
            pltpu.SemaphoreType.DMA,
        ],
    )(efeat, dst)


def kernel(node_features, edge_features, flow_features, edge_index, params):
    src = edge_index[0]
    dst = edge_index[1]
    fpad = jnp.pad(flow_features, ((0, 0), (0, D - DF)))
    nfeat = node_features
    efeat = edge_features
    for i in range(P):
        ep = params["edge"][i]
        w1 = ep["w1"]
        w1e = w1[0:D]
        w1s = w1[D:2 * D]
        w1d = w1[2 * D:3 * D]
        w1fs = jnp.pad(w1[3 * D:3 * D + DF], ((0, D - DF), (0, 0)))
        w1fd = jnp.pad(w1[3 * D + DF:], ((0, D - DF), (0, 0)))
        gs, gd = _tables(nfeat, fpad, w1s, w1d, w1fs, w1fd,
                         ep["b1"].reshape(1, D))
        x = _sc_gather(gs, gd, src, dst)
        efeat = _edge_mlp(efeat, x, w1e, ep["w2"],
                          ep["g"].reshape(1, D), ep["be"].reshape(1, D),
                          ep["b2"].reshape(1, D))
        p = _sc_scatter(efeat, dst).reshape(2, NPAD, D)

        np_ = params["node"][i]
        nw1 = np_["w1"]
        wn = nw1[0:D]
        wa = nw1[D:2 * D]
        wf = jnp.pad(nw1[2 * D:], ((0, D - DF), (0, 0)))
        nfeat = _node_mlp(nfeat, p, fpad, wn, wa, wf,
                          np_["b1"].reshape(1, D), np_["g"].reshape(1, D),
                          np_["be"].reshape(1, D), np_["w2"],
                          np_["b2"].reshape(1, D))
    return nfeat


# speedup vs baseline: 3.8553x; 1.0871x over previous
import jax
import jax.numpy as jnp
from jax import lax
from jax.experimental import pallas as pl
from jax.experimental.pallas import tpu as pltpu
from jax.experimental.pallas import tpu_sc as plsc

N = 10000
E = 320000
D = 128
DF = 16
P = 2

NC = 2
NS = 16
NW = NC * NS

CCH_G = 256
KIDX_G = 128
KC_G = CCH_G // KIDX_G
NB_G = 3
NCH_G = E // CCH_G
NJ_G = (NCH_G + NW - 1) // NW

NPAD = 10240
SLAB = NPAD // NS
CCH_S = 160
KIDX_S = 80
KC_S = CCH_S // KIDX_S
NB_S = 2
NCH_S = E // CCH_S
NJ_S = (NCH_S + NW - 1) // NW

BE = 512
GE = E // BE
BN = 1000
GN = N // BN

_F32 = jnp.float32


def _ln_silu(h, g, b):
    m = jnp.mean(h, axis=-1, keepdims=True)
    c = h - m
    v = jnp.mean(c * c, axis=-1, keepdims=True)
    hn = c * lax.rsqrt(v + 1e-5) * g + b
    return hn * jax.nn.sigmoid(hn)


def _dot(a, b):
    return jnp.dot(a, b, preferred_element_type=_F32)


def _tables_body(n_ref, f_ref, ws_ref, wd_ref, wfs_ref, wfd_ref, b1_ref,
                 gs_ref, gd_ref):
    n = n_ref[...]
    f = f_ref[...]
    gs_ref[...] = _dot(n, ws_ref[...]) + _dot(f, wfs_ref[...]) + b1_ref[...]
    gd_ref[...] = _dot(n, wd_ref[...]) + _dot(f, wfd_ref[...])


def _tables(nfeat, fpad, ws, wd, wfs, wfd, b1):
    row = pl.BlockSpec((BN, D), lambda i: (i, 0))
    wsp = pl.BlockSpec((D, D), lambda i: (0, 0))
    vsp = pl.BlockSpec((1, D), lambda i: (0, 0))
    return pl.pallas_call(
        _tables_body,
        grid=(GN,),
        in_specs=[row, row, wsp, wsp, wsp, wsp, vsp],
        out_specs=[row, row],
        out_shape=[jax.ShapeDtypeStruct((N, D), _F32),
                   jax.ShapeDtypeStruct((N, D), _F32)],
    )(nfeat, fpad, ws, wd, wfs, wfd, b1)


def _edge_body(e_ref, x_ref, w1_ref, w2_ref, lg_ref, lb_ref, b2_ref, o_ref):
    e = e_ref[...]
    h = _dot(e, w1_ref[...]) + x_ref[...]
    a = _ln_silu(h, lg_ref[...], lb_ref[...])
    o_ref[...] = _dot(a, w2_ref[...]) + b2_ref[...] + e


def _edge_mlp(efeat, x, w1e, w2, lg, lb, b2):
    row = pl.BlockSpec((BE, D), lambda i: (i, 0))
    wsp = pl.BlockSpec((D, D), lambda i: (0, 0))
    vsp = pl.BlockSpec((1, D), lambda i: (0, 0))
    return pl.pallas_call(
        _edge_body,
        grid=(GE,),
        in_specs=[row, row, wsp, wsp, vsp, vsp, vsp],
        out_specs=row,
        out_shape=jax.ShapeDtypeStruct((E, D), _F32),
    )(efeat, x, w1e, w2, lg, lb, b2)


def _node_body(n_ref, p_ref, f_ref, wn_ref, wa_ref, wf_ref, b1_ref,
               lg_ref, lb_ref, w2_ref, b2_ref, o_ref):
    n = n_ref[...]
    agg = p_ref[0] + p_ref[1]
    h = _dot(n, wn_ref[...]) + _dot(agg, wa_ref[...]) + _dot(f_ref[...], wf_ref[...]) + b1_ref[...]
    a = _ln_silu(h, lg_ref[...], lb_ref[...])
    o_ref[...] = _dot(a, w2_ref[...]) + b2_ref[...] + n


def _node_mlp(nfeat, p, fpad, wn, wa, wf, b1, lg, lb, w2, b2):
    row = pl.BlockSpec((BN, D), lambda i: (i, 0))
    psp = pl.BlockSpec((2, BN, D), lambda i: (0, i, 0))
    wsp = pl.BlockSpec((D, D), lambda i: (0, 0))
    vsp = pl.BlockSpec((1, D), lambda i: (0, 0))
    return pl.pallas_call(
        _node_body,
        grid=(GN,),
        in_specs=[row, psp, row, wsp, wsp, wsp, vsp, vsp, vsp, wsp, vsp],
        out_specs=row,
        out_shape=jax.ShapeDtypeStruct((N, D), _F32),
    )(nfeat, p, fpad, wn, wa, wf, b1, lg, lb, w2, b2)


def _sc_mesh():
    return plsc.VectorSubcoreMesh(
        core_axis_name="c", subcore_axis_name="s", num_cores=NC, num_subcores=NS)


def _sc_gather_body(gs_hbm, gd_hbm, src_hbm, dst_hbm, out_hbm,
                    idx_s, idx_d, rows,
                    sg0, sg1, sg2, sd0, sd1, sd2, so0, so1, so2):
    sg = (sg0, sg1, sg2)
    sd = (sd0, sd1, sd2)
    so = (so0, so1, so2)
    w = lax.axis_index("s") * NC + lax.axis_index("c")

    def rows_at(b):
        return rows.at[pl.ds(b * CCH_G, CCH_G)]

    def drain(dst_ref, sem):
        pltpu.make_async_copy(gs_hbm.at[pl.ds(0, CCH_G)], dst_ref, sem).wait()

    def slot(j, b):
        j3 = j - 2
        c3 = w + j3 * NW
        b3 = (b - 2) % NB_G

        @pl.when(jnp.logical_and(j3 >= 0, c3 < NCH_G))
        def _():
            drain(rows_at(b3), sd[b3])
            pltpu.async_copy(rows_at(b3), out_hbm.at[pl.ds(c3 * CCH_G, CCH_G)],
                             so[b3])

        j2 = j - 1
        c2 = w + j2 * NW
        b2 = (b - 1) % NB_G

        @pl.when(jnp.logical_and(j2 >= 0, c2 < NCH_G))
        def _():
            drain(rows_at(b2), sg[b2])
            for k in range(KC_G):
                sl = pl.ds(k * KIDX_G, KIDX_G)
                pltpu.async_copy(
                    gd_hbm.at[idx_d.at[pl.ds(b2 * CCH_G + k * KIDX_G, KIDX_G)]],
                    rows.at[pl.ds(b2 * CCH_G + k * KIDX_G, KIDX_G)],
                    sd[b2], add=True)

        c1 = w + j * NW

        @pl.when(c1 < NCH_G)
        def _():
            @pl.when(j >= NB_G)
            def _():
                pltpu.make_async_copy(rows_at(b),
                                      out_hbm.at[pl.ds(0, CCH_G)], so[b]).wait()

            base = c1 * CCH_G
            pltpu.sync_copy(src_hbm.at[pl.ds(base, CCH_G)],
                            idx_s.at[pl.ds(b * CCH_G, CCH_G)])
            pltpu.sync_copy(dst_hbm.at[pl.ds(base, CCH_G)],
                            idx_d.at[pl.ds(b * CCH_G, CCH_G)])
            for k in range(KC_G):
                sl = pl.ds(k * KIDX_G, KIDX_G)
                pltpu.async_copy(
                    gs_hbm.at[idx_s.at[pl.ds(b * CCH_G + k * KIDX_G, KIDX_G)]],
                    rows.at[pl.ds(b * CCH_G + k * KIDX_G, KIDX_G)],
                    sg[b])

    NSLOT = NJ_G + 2
    NGRP = (NSLOT + NB_G - 1) // NB_G

    def group(g, carry):
        for b in range(NB_G):
            slot(g * NB_G + b, b)
        return carry

    lax.fori_loop(0, NGRP, group, 0)
    for b in range(NB_G):
        pltpu.make_async_copy(rows_at(b), out_hbm.at[pl.ds(0, CCH_G)],
                              so[b]).wait()


def _sc_gather(gs, gd, src, dst):
    return pl.kernel(
        _sc_gather_body,
        out_type=jax.ShapeDtypeStruct((E, D), _F32),
        mesh=_sc_mesh(),
        scratch_types=[
            pltpu.VMEM((NB_G * CCH_G,), jnp.int32),
            pltpu.VMEM((NB_G * CCH_G,), jnp.int32),
            pltpu.VMEM((NB_G * CCH_G, D), _F32),
        ] + [pltpu.SemaphoreType.DMA] * 9,
    )(gs, gd, src, dst)


def _sc_scatter_body(e_hbm, dst_hbm, out_hbm, idx2, rows, acc,
                     se0, se1, sa0, sa1):
    se = (se0, se1)
    sa = (sa0, sa1)
    cid = lax.axis_index("c")
    sid = lax.axis_index("s")
    w = sid * NC + cid
    zero = jnp.zeros((16,), _F32)

    def zrow(i, carry):
        for j in range(D // 16):
            rows[i, pl.ds(j * 16, 16)] = zero
        return carry

    lax.fori_loop(0, NB_S * CCH_S, zrow, 0)
    z0 = sid * SLAB
    pltpu.sync_copy(rows, acc.at[pl.ds(z0, NB_S * CCH_S)])
    pltpu.sync_copy(rows, acc.at[pl.ds(z0 + NB_S * CCH_S, NB_S * CCH_S)])
    plsc.subcore_barrier()

    def rows_at(b):
        return rows.at[pl.ds(b * CCH_S, CCH_S)]

    def slot(j, b):
        j2 = j - 1
        c2 = w + j2 * NW
        b2 = (b - 1) % NB_S

        @pl.when(jnp.logical_and(j2 >= 0, c2 < NCH_S))
        def _():
            pltpu.make_async_copy(e_hbm.at[pl.ds(0, CCH_S)], rows_at(b2),
                                  se[b2]).wait()
            for k in range(KC_S):
                pltpu.async_copy(
                    rows.at[pl.ds(b2 * CCH_S + k * KIDX_S, KIDX_S)],
                    acc.at[idx2.at[b2 * KC_S + k]], sa[b2], add=True)

        c1 = w + j * NW

        @pl.when(c1 < NCH_S)
        def _():
            @pl.when(j >= NB_S)
            def _():
                for k in range(KC_S):
                    pltpu.make_async_copy(
                        rows.at[pl.ds(b * CCH_S + k * KIDX_S, KIDX_S)],
                        acc.at[pl.ds(0, KIDX_S)], sa[b]).wait()

            base = c1 * CCH_S
            for k in range(KC_S):
                pltpu.sync_copy(dst_hbm.at[pl.ds(base + k * KIDX_S, KIDX_S)],
                                idx2.at[b * KC_S + k])
            pltpu.async_copy(e_hbm.at[pl.ds(base, CCH_S)], rows_at(b), se[b])

    NSLOT = NJ_S + 1
    NGRP = (NSLOT + NB_S - 1) // NB_S

    def group(g, carry):
        for b in range(NB_S):
            slot(g * NB_S + b, b)
        return carry

    lax.fori_loop(0, NGRP, group, 0)
    for b in range(NB_S):
        for k in range(KC_S):
            pltpu.make_async_copy(
                rows.at[pl.ds(b * CCH_S + k * KIDX_S, KIDX_S)],
                acc.at[pl.ds(0, KIDX_S)], sa[b]).wait()
    plsc.subcore_barrier()
    pltpu.sync_copy(acc.at[pl.ds(sid * SLAB, SLAB)],
                    out_hbm.at[pl.ds(cid * NPAD + sid * SLAB, SLAB)])


def _sc_scatter(efeat, dst):
    return pl.kernel(
        _sc_scatter_body,
        out_type=jax.ShapeDtypeStruct((2 * NPAD, D), _F32),
        mesh=_sc_mesh(),
        scratch_types=[
            pltpu.VMEM((NB_S * KC_S, KIDX_S), jnp.int32),
            pltpu.VMEM((NB_S * CCH_S, D), _F32),
            pltpu.VMEM_SHARED((NPAD, D), _F32),
        ] + [pltpu.SemaphoreType.DMA] * 4,
    )(efeat, dst)


def kernel(node_features, edge_features, flow_features, edge_index, params):
    src = edge_index[0]
    dst = edge_index[1]
    fpad = jnp.pad(flow_features, ((0, 0), (0, D - DF)))
    nfeat = node_features
    efeat = edge_features
    for i in range(P):
        ep = params["edge"][i]
        w1 = ep["w1"]
        w1e = w1[0:D]
        w1s = w1[D:2 * D]
        w1d = w1[2 * D:3 * D]
        w1fs = jnp.pad(w1[3 * D:3 * D + DF], ((0, D - DF), (0, 0)))
        w1fd = jnp.pad(w1[3 * D + DF:], ((0, D - DF), (0, 0)))
        gs, gd = _tables(nfeat, fpad, w1s, w1d, w1fs, w1fd,
                         ep["b1"].reshape(1, D))
        x = _sc_gather(gs, gd, src, dst)
        efeat = _edge_mlp(efeat, x, w1e, ep["w2"],
                          ep["g"].reshape(1, D), ep["be"].reshape(1, D),
                          ep["b2"].reshape(1, D))
        p = _sc_scatter(efeat, dst).reshape(2, NPAD, D)

        np_ = params["node"][i]
        nw1 = np_["w1"]
        wn = nw1[0:D]
        wa = nw1[D:2 * D]
        wf = jnp.pad(nw1[2 * D:], ((0, D - DF), (0, 0)))
        nfeat = _node_mlp(nfeat, p, fpad, wn, wa, wf,
                          np_["b1"].reshape(1, D), np_["g"].reshape(1, D),
                          np_["be"].reshape(1, D), np_["w2"],
                          np_["b2"].reshape(1, D))
    return nfeat


# speedup vs baseline: 4.6128x; 1.1965x over previous
import jax
import jax.numpy as jnp
from jax import lax
from jax.experimental import pallas as pl
from jax.experimental.pallas import tpu as pltpu
from jax.experimental.pallas import tpu_sc as plsc

N = 10000
E = 320000
D = 128
DF = 16
P = 2

NC = 2
NS = 16
NW = NC * NS

CCH_G = 256
KIDX_G = 128
KC_G = CCH_G // KIDX_G
NB_G = 3
NCH_G = E // CCH_G
NJ_G = (NCH_G + NW - 1) // NW

NPAD = 10240
SLAB = NPAD // NS
CCH_S = 160
KIDX_S = 80
KC_S = CCH_S // KIDX_S
NB_S = 2
NCH_S = E // CCH_S
NJ_S = (NCH_S + NW - 1) // NW

BE = 1000
GE = E // BE
BN = 1000
GN = N // BN

_F32 = jnp.float32


def _ln_silu(h, g, b):
    m = jnp.mean(h, axis=-1, keepdims=True)
    c = h - m
    v = jnp.mean(c * c, axis=-1, keepdims=True)
    hn = c * lax.rsqrt(v + 1e-5) * g + b
    return hn * jax.nn.sigmoid(hn)


def _dot(a, b):
    return jnp.dot(a.astype(jnp.bfloat16), b.astype(jnp.bfloat16),
                   preferred_element_type=_F32)


def _tables_body(n_ref, f_ref, ws_ref, wd_ref, wfs_ref, wfd_ref, b1_ref,
                 gs_ref, gd_ref):
    n = n_ref[...]
    f = f_ref[...]
    gs_ref[...] = _dot(n, ws_ref[...]) + _dot(f, wfs_ref[...]) + b1_ref[...]
    gd_ref[...] = _dot(n, wd_ref[...]) + _dot(f, wfd_ref[...])


def _tables(nfeat, fpad, ws, wd, wfs, wfd, b1):
    row = pl.BlockSpec((BN, D), lambda i: (i, 0))
    wsp = pl.BlockSpec((D, D), lambda i: (0, 0))
    vsp = pl.BlockSpec((1, D), lambda i: (0, 0))
    return pl.pallas_call(
        _tables_body,
        grid=(GN,),
        in_specs=[row, row, wsp, wsp, wsp, wsp, vsp],
        out_specs=[row, row],
        out_shape=[jax.ShapeDtypeStruct((N, D), _F32),
                   jax.ShapeDtypeStruct((N, D), _F32)],
    )(nfeat, fpad, ws, wd, wfs, wfd, b1)


def _edge_body(e_ref, x_ref, w1_ref, w2_ref, lg_ref, lb_ref, b2_ref, o_ref):
    e = e_ref[...]
    h = _dot(e, w1_ref[...]) + x_ref[...]
    a = _ln_silu(h, lg_ref[...], lb_ref[...])
    o_ref[...] = _dot(a, w2_ref[...]) + b2_ref[...] + e


def _edge_mlp(efeat, x, w1e, w2, lg, lb, b2):
    row = pl.BlockSpec((BE, D), lambda i: (i, 0))
    wsp = pl.BlockSpec((D, D), lambda i: (0, 0))
    vsp = pl.BlockSpec((1, D), lambda i: (0, 0))
    return pl.pallas_call(
        _edge_body,
        grid=(GE,),
        in_specs=[row, row, wsp, wsp, vsp, vsp, vsp],
        out_specs=row,
        out_shape=jax.ShapeDtypeStruct((E, D), _F32),
    )(efeat, x, w1e, w2, lg, lb, b2)


def _node_body(n_ref, p_ref, f_ref, wn_ref, wa_ref, wf_ref, b1_ref,
               lg_ref, lb_ref, w2_ref, b2_ref, o_ref):
    n = n_ref[...]
    agg = p_ref[0] + p_ref[1]
    h = _dot(n, wn_ref[...]) + _dot(agg, wa_ref[...]) + _dot(f_ref[...], wf_ref[...]) + b1_ref[...]
    a = _ln_silu(h, lg_ref[...], lb_ref[...])
    o_ref[...] = _dot(a, w2_ref[...]) + b2_ref[...] + n


def _node_mlp(nfeat, p, fpad, wn, wa, wf, b1, lg, lb, w2, b2):
    row = pl.BlockSpec((BN, D), lambda i: (i, 0))
    psp = pl.BlockSpec((2, BN, D), lambda i: (0, i, 0))
    wsp = pl.BlockSpec((D, D), lambda i: (0, 0))
    vsp = pl.BlockSpec((1, D), lambda i: (0, 0))
    return pl.pallas_call(
        _node_body,
        grid=(GN,),
        in_specs=[row, psp, row, wsp, wsp, wsp, vsp, vsp, vsp, wsp, vsp],
        out_specs=row,
        out_shape=jax.ShapeDtypeStruct((N, D), _F32),
    )(nfeat, p, fpad, wn, wa, wf, b1, lg, lb, w2, b2)


def _sc_mesh():
    return plsc.VectorSubcoreMesh(
        core_axis_name="c", subcore_axis_name="s", num_cores=NC, num_subcores=NS)


def _sc_gather_body(gs_hbm, gd_hbm, src_hbm, dst_hbm, out_hbm,
                    idx_s, idx_d, rows,
                    sg0, sg1, sg2, sd0, sd1, sd2, so0, so1, so2):
    sg = (sg0, sg1, sg2)
    sd = (sd0, sd1, sd2)
    so = (so0, so1, so2)
    w = lax.axis_index("s") * NC + lax.axis_index("c")

    def rows_at(b):
        return rows.at[pl.ds(b * CCH_G, CCH_G)]

    def drain(dst_ref, sem):
        pltpu.make_async_copy(gs_hbm.at[pl.ds(0, CCH_G)], dst_ref, sem).wait()

    def slot(j, b):
        j3 = j - 2
        c3 = w + j3 * NW
        b3 = (b - 2) % NB_G

        @pl.when(jnp.logical_and(j3 >= 0, c3 < NCH_G))
        def _():
            drain(rows_at(b3), sd[b3])
            pltpu.async_copy(rows_at(b3), out_hbm.at[pl.ds(c3 * CCH_G, CCH_G)],
                             so[b3])

        j2 = j - 1
        c2 = w + j2 * NW
        b2 = (b - 1) % NB_G

        @pl.when(jnp.logical_and(j2 >= 0, c2 < NCH_G))
        def _():
            drain(rows_at(b2), sg[b2])
            for k in range(KC_G):
                sl = pl.ds(k * KIDX_G, KIDX_G)
                pltpu.async_copy(
                    gd_hbm.at[idx_d.at[pl.ds(b2 * CCH_G + k * KIDX_G, KIDX_G)]],
                    rows.at[pl.ds(b2 * CCH_G + k * KIDX_G, KIDX_G)],
                    sd[b2], add=True)

        c1 = w + j * NW

        @pl.when(c1 < NCH_G)
        def _():
            @pl.when(j >= NB_G)
            def _():
                pltpu.make_async_copy(rows_at(b),
                                      out_hbm.at[pl.ds(0, CCH_G)], so[b]).wait()

            base = c1 * CCH_G
            pltpu.sync_copy(src_hbm.at[pl.ds(base, CCH_G)],
                            idx_s.at[pl.ds(b * CCH_G, CCH_G)])
            pltpu.sync_copy(dst_hbm.at[pl.ds(base, CCH_G)],
                            idx_d.at[pl.ds(b * CCH_G, CCH_G)])
            for k in range(KC_G):
                sl = pl.ds(k * KIDX_G, KIDX_G)
                pltpu.async_copy(
                    gs_hbm.at[idx_s.at[pl.ds(b * CCH_G + k * KIDX_G, KIDX_G)]],
                    rows.at[pl.ds(b * CCH_G + k * KIDX_G, KIDX_G)],
                    sg[b])

    NSLOT = NJ_G + 2
    NGRP = (NSLOT + NB_G - 1) // NB_G

    def group(g, carry):
        for b in range(NB_G):
            slot(g * NB_G + b, b)
        return carry

    lax.fori_loop(0, NGRP, group, 0)
    for b in range(NB_G):
        pltpu.make_async_copy(rows_at(b), out_hbm.at[pl.ds(0, CCH_G)],
                              so[b]).wait()


def _sc_gather(gs, gd, src, dst):
    return pl.kernel(
        _sc_gather_body,
        out_type=jax.ShapeDtypeStruct((E, D), _F32),
        mesh=_sc_mesh(),
        scratch_types=[
            pltpu.VMEM((NB_G * CCH_G,), jnp.int32),
            pltpu.VMEM((NB_G * CCH_G,), jnp.int32),
            pltpu.VMEM((NB_G * CCH_G, D), _F32),
        ] + [pltpu.SemaphoreType.DMA] * 9,
    )(gs, gd, src, dst)


def _sc_scatter_body(e_hbm, dst_hbm, out_hbm, idx2, rows, acc,
                     se0, se1, sa0, sa1):
    se = (se0, se1)
    sa = (sa0, sa1)
    cid = lax.axis_index("c")
    sid = lax.axis_index("s")
    w = sid * NC + cid
    zero = jnp.zeros((16,), _F32)

    def zrow(i, carry):
        for j in range(D // 16):
            rows[i, pl.ds(j * 16, 16)] = zero
        return carry

    lax.fori_loop(0, NB_S * CCH_S, zrow, 0)
    z0 = sid * SLAB
    pltpu.sync_copy(rows, acc.at[pl.ds(z0, NB_S * CCH_S)])
    pltpu.sync_copy(rows, acc.at[pl.ds(z0 + NB_S * CCH_S, NB_S * CCH_S)])
    plsc.subcore_barrier()

    def rows_at(b):
        return rows.at[pl.ds(b * CCH_S, CCH_S)]

    def slot(j, b):
        j2 = j - 1
        c2 = w + j2 * NW
        b2 = (b - 1) % NB_S

        @pl.when(jnp.logical_and(j2 >= 0, c2 < NCH_S))
        def _():
            pltpu.make_async_copy(e_hbm.at[pl.ds(0, CCH_S)], rows_at(b2),
                                  se[b2]).wait()
            for k in range(KC_S):
                pltpu.async_copy(
                    rows.at[pl.ds(b2 * CCH_S + k * KIDX_S, KIDX_S)],
                    acc.at[idx2.at[b2 * KC_S + k]], sa[b2], add=True)

        c1 = w + j * NW

        @pl.when(c1 < NCH_S)
        def _():
            @pl.when(j >= NB_S)
            def _():
                for k in range(KC_S):
                    pltpu.make_async_copy(
                        rows.at[pl.ds(b * CCH_S + k * KIDX_S, KIDX_S)],
                        acc.at[pl.ds(0, KIDX_S)], sa[b]).wait()

            base = c1 * CCH_S
            for k in range(KC_S):
                pltpu.sync_copy(dst_hbm.at[pl.ds(base + k * KIDX_S, KIDX_S)],
                                idx2.at[b * KC_S + k])
            pltpu.async_copy(e_hbm.at[pl.ds(base, CCH_S)], rows_at(b), se[b])

    NSLOT = NJ_S + 1
    NGRP = (NSLOT + NB_S - 1) // NB_S

    def group(g, carry):
        for b in range(NB_S):
            slot(g * NB_S + b, b)
        return carry

    lax.fori_loop(0, NGRP, group, 0)
    for b in range(NB_S):
        for k in range(KC_S):
            pltpu.make_async_copy(
                rows.at[pl.ds(b * CCH_S + k * KIDX_S, KIDX_S)],
                acc.at[pl.ds(0, KIDX_S)], sa[b]).wait()
    plsc.subcore_barrier()
    pltpu.sync_copy(acc.at[pl.ds(sid * SLAB, SLAB)],
                    out_hbm.at[pl.ds(cid * NPAD + sid * SLAB, SLAB)])


def _sc_scatter(efeat, dst):
    return pl.kernel(
        _sc_scatter_body,
        out_type=jax.ShapeDtypeStruct((2 * NPAD, D), _F32),
        mesh=_sc_mesh(),
        scratch_types=[
            pltpu.VMEM((NB_S * KC_S, KIDX_S), jnp.int32),
            pltpu.VMEM((NB_S * CCH_S, D), _F32),
            pltpu.VMEM_SHARED((NPAD, D), _F32),
        ] + [pltpu.SemaphoreType.DMA] * 4,
    )(efeat, dst)


def kernel(node_features, edge_features, flow_features, edge_index, params):
    src = edge_index[0]
    dst = edge_index[1]
    fpad = jnp.pad(flow_features, ((0, 0), (0, D - DF)))
    nfeat = node_features
    efeat = edge_features
    for i in range(P):
        ep = params["edge"][i]
        w1 = ep["w1"]
        w1e = w1[0:D]
        w1s = w1[D:2 * D]
        w1d = w1[2 * D:3 * D]
        w1fs = jnp.pad(w1[3 * D:3 * D + DF], ((0, D - DF), (0, 0)))
        w1fd = jnp.pad(w1[3 * D + DF:], ((0, D - DF), (0, 0)))
        gs, gd = _tables(nfeat, fpad, w1s, w1d, w1fs, w1fd,
                         ep["b1"].reshape(1, D))
        x = _sc_gather(gs, gd, src, dst)
        efeat = _edge_mlp(efeat, x, w1e, ep["w2"],
                          ep["g"].reshape(1, D), ep["be"].reshape(1, D),
                          ep["b2"].reshape(1, D))
        p = _sc_scatter(efeat, dst).reshape(2, NPAD, D)

        np_ = params["node"][i]
        nw1 = np_["w1"]
        wn = nw1[0:D]
        wa = nw1[D:2 * D]
        wf = jnp.pad(nw1[2 * D:], ((0, D - DF), (0, 0)))
        nfeat = _node_mlp(nfeat, p, fpad, wn, wa, wf,
                          np_["b1"].reshape(1, D), np_["g"].reshape(1, D),
                          np_["be"].reshape(1, D), np_["w2"],
                          np_["b2"].reshape(1, D))
    return nfeat


# speedup vs baseline: 4.8864x; 1.0593x over previous
import jax
import jax.numpy as jnp
from jax import lax
from jax.experimental import pallas as pl
from jax.experimental.pallas import tpu as pltpu
from jax.experimental.pallas import tpu_sc as plsc

N = 10000
E = 320000
D = 128
DF = 16
P = 2

NC = 2
NS = 16
NW = NC * NS

NSPLIT = 2
EH = E // NSPLIT

CCH_G = 256
KIDX_G = 128
KC_G = CCH_G // KIDX_G
NB_G = 3
NCH_G = EH // CCH_G
NJ_G = (NCH_G + NW - 1) // NW

NPAD = 10240
SLAB = NPAD // NS
CCH_S = 160
KIDX_S = 80
KC_S = CCH_S // KIDX_S
NB_S = 2
NCH_S = EH // CCH_S
NJ_S = (NCH_S + NW - 1) // NW

BE = 1000
GE = EH // BE
BN = 1000
GN = N // BN

_F32 = jnp.float32


def _ln_silu(h, g, b):
    m = jnp.mean(h, axis=-1, keepdims=True)
    c = h - m
    v = jnp.mean(c * c, axis=-1, keepdims=True)
    hn = c * lax.rsqrt(v + 1e-5) * g + b
    return hn * jax.nn.sigmoid(hn)


def _dot(a, b):
    return jnp.dot(a.astype(jnp.bfloat16), b.astype(jnp.bfloat16),
                   preferred_element_type=_F32)


def _tables_body(n_ref, f_ref, ws_ref, wd_ref, wfs_ref, wfd_ref, b1_ref,
                 gs_ref, gd_ref):
    n = n_ref[...]
    f = f_ref[...]
    gs_ref[...] = _dot(n, ws_ref[...]) + _dot(f, wfs_ref[...]) + b1_ref[...]
    gd_ref[...] = _dot(n, wd_ref[...]) + _dot(f, wfd_ref[...])


def _tables(nfeat, fpad, ws, wd, wfs, wfd, b1):
    row = pl.BlockSpec((BN, D), lambda i: (i, 0))
    wsp = pl.BlockSpec((D, D), lambda i: (0, 0))
    vsp = pl.BlockSpec((1, D), lambda i: (0, 0))
    return pl.pallas_call(
        _tables_body,
        grid=(GN,),
        in_specs=[row, row, wsp, wsp, wsp, wsp, vsp],
        out_specs=[row, row],
        out_shape=[jax.ShapeDtypeStruct((N, D), _F32),
                   jax.ShapeDtypeStruct((N, D), _F32)],
    )(nfeat, fpad, ws, wd, wfs, wfd, b1)


def _edge_body(e_ref, x_ref, w1_ref, w2_ref, lg_ref, lb_ref, b2_ref, o_ref):
    e = e_ref[...]
    h = _dot(e, w1_ref[...]) + x_ref[...]
    a = _ln_silu(h, lg_ref[...], lb_ref[...])
    o_ref[...] = _dot(a, w2_ref[...]) + b2_ref[...] + e


def _edge_mlp(efeat, x, w1e, w2, lg, lb, b2):
    row = pl.BlockSpec((BE, D), lambda i: (i, 0))
    wsp = pl.BlockSpec((D, D), lambda i: (0, 0))
    vsp = pl.BlockSpec((1, D), lambda i: (0, 0))
    return pl.pallas_call(
        _edge_body,
        grid=(GE,),
        in_specs=[row, row, wsp, wsp, vsp, vsp, vsp],
        out_specs=row,
        out_shape=jax.ShapeDtypeStruct((EH, D), _F32),
    )(efeat, x, w1e, w2, lg, lb, b2)


def _node_body(n_ref, p_ref, f_ref, wn_ref, wa_ref, wf_ref, b1_ref,
               lg_ref, lb_ref, w2_ref, b2_ref, o_ref):
    n = n_ref[...]
    agg = p_ref[0] + p_ref[1] + p_ref[2] + p_ref[3]
    h = _dot(n, wn_ref[...]) + _dot(agg, wa_ref[...]) + _dot(f_ref[...], wf_ref[...]) + b1_ref[...]
    a = _ln_silu(h, lg_ref[...], lb_ref[...])
    o_ref[...] = _dot(a, w2_ref[...]) + b2_ref[...] + n


def _node_mlp(nfeat, p, fpad, wn, wa, wf, b1, lg, lb, w2, b2):
    row = pl.BlockSpec((BN, D), lambda i: (i, 0))
    psp = pl.BlockSpec((2 * NSPLIT, BN, D), lambda i: (0, i, 0))
    wsp = pl.BlockSpec((D, D), lambda i: (0, 0))
    vsp = pl.BlockSpec((1, D), lambda i: (0, 0))
    return pl.pallas_call(
        _node_body,
        grid=(GN,),
        in_specs=[row, psp, row, wsp, wsp, wsp, vsp, vsp, vsp, wsp, vsp],
        out_specs=row,
        out_shape=jax.ShapeDtypeStruct((N, D), _F32),
    )(nfeat, p, fpad, wn, wa, wf, b1, lg, lb, w2, b2)


def _sc_mesh():
    return plsc.VectorSubcoreMesh(
        core_axis_name="c", subcore_axis_name="s", num_cores=NC, num_subcores=NS)


def _sc_gather_body(gs_hbm, gd_hbm, src_hbm, dst_hbm, out_hbm,
                    idx_s, idx_d, rows,
                    sg0, sg1, sg2, sd0, sd1, sd2, so0, so1, so2):
    sg = (sg0, sg1, sg2)
    sd = (sd0, sd1, sd2)
    so = (so0, so1, so2)
    w = lax.axis_index("s") * NC + lax.axis_index("c")

    def rows_at(b):
        return rows.at[pl.ds(b * CCH_G, CCH_G)]

    def drain(dst_ref, sem):
        pltpu.make_async_copy(gs_hbm.at[pl.ds(0, CCH_G)], dst_ref, sem).wait()

    def slot(j, b):
        j3 = j - 2
        c3 = w + j3 * NW
        b3 = (b - 2) % NB_G

        @pl.when(jnp.logical_and(j3 >= 0, c3 < NCH_G))
        def _():
            drain(rows_at(b3), sd[b3])
            pltpu.async_copy(rows_at(b3), out_hbm.at[pl.ds(c3 * CCH_G, CCH_G)],
                             so[b3])

        j2 = j - 1
        c2 = w + j2 * NW
        b2 = (b - 1) % NB_G

        @pl.when(jnp.logical_and(j2 >= 0, c2 < NCH_G))
        def _():
            drain(rows_at(b2), sg[b2])
            for k in range(KC_G):
                sl = pl.ds(k * KIDX_G, KIDX_G)
                pltpu.async_copy(
                    gd_hbm.at[idx_d.at[pl.ds(b2 * CCH_G + k * KIDX_G, KIDX_G)]],
                    rows.at[pl.ds(b2 * CCH_G + k * KIDX_G, KIDX_G)],
                    sd[b2], add=True)

        c1 = w + j * NW

        @pl.when(c1 < NCH_G)
        def _():
            @pl.when(j >= NB_G)
            def _():
                pltpu.make_async_copy(rows_at(b),
                                      out_hbm.at[pl.ds(0, CCH_G)], so[b]).wait()

            base = c1 * CCH_G
            pltpu.sync_copy(src_hbm.at[pl.ds(base, CCH_G)],
                            idx_s.at[pl.ds(b * CCH_G, CCH_G)])
            pltpu.sync_copy(dst_hbm.at[pl.ds(base, CCH_G)],
                            idx_d.at[pl.ds(b * CCH_G, CCH_G)])
            for k in range(KC_G):
                sl = pl.ds(k * KIDX_G, KIDX_G)
                pltpu.async_copy(
                    gs_hbm.at[idx_s.at[pl.ds(b * CCH_G + k * KIDX_G, KIDX_G)]],
                    rows.at[pl.ds(b * CCH_G + k * KIDX_G, KIDX_G)],
                    sg[b])

    NSLOT = NJ_G + 2
    NGRP = (NSLOT + NB_G - 1) // NB_G

    def group(g, carry):
        for b in range(NB_G):
            slot(g * NB_G + b, b)
        return carry

    lax.fori_loop(0, NGRP, group, 0)
    for b in range(NB_G):
        pltpu.make_async_copy(rows_at(b), out_hbm.at[pl.ds(0, CCH_G)],
                              so[b]).wait()


def _sc_gather(gs, gd, src, dst):
    return pl.kernel(
        _sc_gather_body,
        out_type=jax.ShapeDtypeStruct((EH, D), _F32),
        mesh=_sc_mesh(),
        scratch_types=[
            pltpu.VMEM((NB_G * CCH_G,), jnp.int32),
            pltpu.VMEM((NB_G * CCH_G,), jnp.int32),
            pltpu.VMEM((NB_G * CCH_G, D), _F32),
        ] + [pltpu.SemaphoreType.DMA] * 9,
    )(gs, gd, src, dst)


def _sc_scatter_body(e_hbm, dst_hbm, out_hbm, idx2, rows, acc,
                     se0, se1, sa0, sa1):
    se = (se0, se1)
    sa = (sa0, sa1)
    cid = lax.axis_index("c")
    sid = lax.axis_index("s")
    w = sid * NC + cid
    zero = jnp.zeros((16,), _F32)

    def zrow(i, carry):
        for j in range(D // 16):
            rows[i, pl.ds(j * 16, 16)] = zero
        return carry

    lax.fori_loop(0, NB_S * CCH_S, zrow, 0)
    z0 = sid * SLAB
    pltpu.sync_copy(rows, acc.at[pl.ds(z0, NB_S * CCH_S)])
    pltpu.sync_copy(rows, acc.at[pl.ds(z0 + NB_S * CCH_S, NB_S * CCH_S)])
    plsc.subcore_barrier()

    def rows_at(b):
        return rows.at[pl.ds(b * CCH_S, CCH_S)]

    def slot(j, b):
        j2 = j - 1
        c2 = w + j2 * NW
        b2 = (b - 1) % NB_S

        @pl.when(jnp.logical_and(j2 >= 0, c2 < NCH_S))
        def _():
            pltpu.make_async_copy(e_hbm.at[pl.ds(0, CCH_S)], rows_at(b2),
                                  se[b2]).wait()
            for k in range(KC_S):
                pltpu.async_copy(
                    rows.at[pl.ds(b2 * CCH_S + k * KIDX_S, KIDX_S)],
                    acc.at[idx2.at[b2 * KC_S + k]], sa[b2], add=True)

        c1 = w + j * NW

        @pl.when(c1 < NCH_S)
        def _():
            @pl.when(j >= NB_S)
            def _():
                for k in range(KC_S):
                    pltpu.make_async_copy(
                        rows.at[pl.ds(b * CCH_S + k * KIDX_S, KIDX_S)],
                        acc.at[pl.ds(0, KIDX_S)], sa[b]).wait()

            base = c1 * CCH_S
            for k in range(KC_S):
                pltpu.sync_copy(dst_hbm.at[pl.ds(base + k * KIDX_S, KIDX_S)],
                                idx2.at[b * KC_S + k])
            pltpu.async_copy(e_hbm.at[pl.ds(base, CCH_S)], rows_at(b), se[b])

    NSLOT = NJ_S + 1
    NGRP = (NSLOT + NB_S - 1) // NB_S

    def group(g, carry):
        for b in range(NB_S):
            slot(g * NB_S + b, b)
        return carry

    lax.fori_loop(0, NGRP, group, 0)
    for b in range(NB_S):
        for k in range(KC_S):
            pltpu.make_async_copy(
                rows.at[pl.ds(b * CCH_S + k * KIDX_S, KIDX_S)],
                acc.at[pl.ds(0, KIDX_S)], sa[b]).wait()
    plsc.subcore_barrier()
    pltpu.sync_copy(acc.at[pl.ds(sid * SLAB, SLAB)],
                    out_hbm.at[pl.ds(cid * NPAD + sid * SLAB, SLAB)])


def _sc_scatter(efeat, dst):
    return pl.kernel(
        _sc_scatter_body,
        out_type=jax.ShapeDtypeStruct((2 * NPAD, D), _F32),
        mesh=_sc_mesh(),
        scratch_types=[
            pltpu.VMEM((NB_S * KC_S, KIDX_S), jnp.int32),
            pltpu.VMEM((NB_S * CCH_S, D), _F32),
            pltpu.VMEM_SHARED((NPAD, D), _F32),
        ] + [pltpu.SemaphoreType.DMA] * 4,
    )(efeat, dst)


def kernel(node_features, edge_features, flow_features, edge_index, params):
    srcs = [edge_index[0, h * EH:(h + 1) * EH] for h in range(NSPLIT)]
    dsts = [edge_index[1, h * EH:(h + 1) * EH] for h in range(NSPLIT)]
    fpad = jnp.pad(flow_features, ((0, 0), (0, D - DF)))
    nfeat = node_features
    efeat = [edge_features[h * EH:(h + 1) * EH] for h in range(NSPLIT)]
    for i in range(P):
        ep = params["edge"][i]
        w1 = ep["w1"]
        w1e = w1[0:D]
        w1s = w1[D:2 * D]
        w1d = w1[2 * D:3 * D]
        w1fs = jnp.pad(w1[3 * D:3 * D + DF], ((0, D - DF), (0, 0)))
        w1fd = jnp.pad(w1[3 * D + DF:], ((0, D - DF), (0, 0)))
        gs, gd = _tables(nfeat, fpad, w1s, w1d, w1fs, w1fd,
                         ep["b1"].reshape(1, D))
        xs = [_sc_gather(gs, gd, srcs[h], dsts[h]) for h in range(NSPLIT)]
        efeat = [_edge_mlp(efeat[h], xs[h], w1e, ep["w2"],
                           ep["g"].reshape(1, D), ep["be"].reshape(1, D),
                           ep["b2"].reshape(1, D)) for h in range(NSPLIT)]
        ps = [_sc_scatter(efeat[h], dsts[h]).reshape(2, NPAD, D)
              for h in range(NSPLIT)]
        p = jnp.concatenate(ps, axis=0)

        np_ = params["node"][i]
        nw1 = np_["w1"]
        wn = nw1[0:D]
        wa = nw1[D:2 * D]
        wf = jnp.pad(nw1[2 * D:], ((0, D - DF), (0, 0)))
        nfeat = _node_mlp(nfeat, p, fpad, wn, wa, wf,
                          np_["b1"].reshape(1, D), np_["g"].reshape(1, D),
                          np_["be"].reshape(1, D), np_["w2"],
                          np_["b2"].reshape(1, D))
    return nfeat


# speedup vs baseline: 5.0983x; 1.0434x over previous
import jax
import jax.numpy as jnp
from jax import lax
from jax.experimental import pallas as pl
from jax.experimental.pallas import tpu as pltpu
from jax.experimental.pallas import tpu_sc as plsc

N = 10000
E = 320000
D = 128
DF = 16
P = 2

NC = 2
NS = 16
NW = NC * NS

NSPLIT = 4
EH = E // NSPLIT

CCH_G = 160
KIDX_G = 80
KC_G = CCH_G // KIDX_G
NB_G = 3
NCH_G = EH // CCH_G
NJ_G = (NCH_G + NW - 1) // NW

NPAD = 10240
SLAB = NPAD // NS
CCH_S = 160
KIDX_S = 80
KC_S = CCH_S // KIDX_S
NB_S = 2
NCH_S = EH // CCH_S
NJ_S = (NCH_S + NW - 1) // NW

BE = 800
GE = EH // BE
BN = 1000
GN = N // BN

_F32 = jnp.float32


def _ln_silu(h, g, b):
    m = jnp.mean(h, axis=-1, keepdims=True)
    c = h - m
    v = jnp.mean(c * c, axis=-1, keepdims=True)
    hn = c * lax.rsqrt(v + 1e-5) * g + b
    return hn * jax.nn.sigmoid(hn)


def _dot(a, b):
    return jnp.dot(a.astype(jnp.bfloat16), b.astype(jnp.bfloat16),
                   preferred_element_type=_F32)


def _tables_body(n_ref, f_ref, ws_ref, wd_ref, wfs_ref, wfd_ref, b1_ref,
                 gs_ref, gd_ref):
    n = n_ref[...]
    f = f_ref[...]
    gs_ref[...] = _dot(n, ws_ref[...]) + _dot(f, wfs_ref[...]) + b1_ref[...]
    gd_ref[...] = _dot(n, wd_ref[...]) + _dot(f, wfd_ref[...])


def _tables(nfeat, fpad, ws, wd, wfs, wfd, b1):
    row = pl.BlockSpec((BN, D), lambda i: (i, 0))
    wsp = pl.BlockSpec((D, D), lambda i: (0, 0))
    vsp = pl.BlockSpec((1, D), lambda i: (0, 0))
    return pl.pallas_call(
        _tables_body,
        grid=(GN,),
        in_specs=[row, row, wsp, wsp, wsp, wsp, vsp],
        out_specs=[row, row],
        out_shape=[jax.ShapeDtypeStruct((N, D), _F32),
                   jax.ShapeDtypeStruct((N, D), _F32)],
    )(nfeat, fpad, ws, wd, wfs, wfd, b1)


def _edge_body(e_ref, x_ref, w1_ref, w2_ref, lg_ref, lb_ref, b2_ref, o_ref):
    e = e_ref[...]
    h = _dot(e, w1_ref[...]) + x_ref[...]
    a = _ln_silu(h, lg_ref[...], lb_ref[...])
    o_ref[...] = _dot(a, w2_ref[...]) + b2_ref[...] + e


def _edge_mlp(efeat, x, w1e, w2, lg, lb, b2):
    row = pl.BlockSpec((BE, D), lambda i: (i, 0))
    xrow = pl.BlockSpec((BE, D), lambda i: (i, 0))
    wsp = pl.BlockSpec((D, D), lambda i: (0, 0))
    vsp = pl.BlockSpec((1, D), lambda i: (0, 0))
    return pl.pallas_call(
        _edge_body,
        grid=(GE,),
        in_specs=[row, xrow, wsp, wsp, vsp, vsp, vsp],
        out_specs=row,
        out_shape=jax.ShapeDtypeStruct((EH, D), _F32),
    )(efeat, x, w1e, w2, lg, lb, b2)


def _node_body(n_ref, p_ref, f_ref, wn_ref, wa_ref, wf_ref, b1_ref,
               lg_ref, lb_ref, w2_ref, b2_ref, o_ref):
    n = n_ref[...]
    agg = p_ref[0] + p_ref[1] + p_ref[2] + p_ref[3] + p_ref[4] + p_ref[5] + p_ref[6] + p_ref[7]
    h = _dot(n, wn_ref[...]) + _dot(agg, wa_ref[...]) + _dot(f_ref[...], wf_ref[...]) + b1_ref[...]
    a = _ln_silu(h, lg_ref[...], lb_ref[...])
    o_ref[...] = _dot(a, w2_ref[...]) + b2_ref[...] + n


def _node_mlp(nfeat, p, fpad, wn, wa, wf, b1, lg, lb, w2, b2):
    row = pl.BlockSpec((BN, D), lambda i: (i, 0))
    psp = pl.BlockSpec((2 * NSPLIT, BN, D), lambda i: (0, i, 0))
    wsp = pl.BlockSpec((D, D), lambda i: (0, 0))
    vsp = pl.BlockSpec((1, D), lambda i: (0, 0))
    return pl.pallas_call(
        _node_body,
        grid=(GN,),
        in_specs=[row, psp, row, wsp, wsp, wsp, vsp, vsp, vsp, wsp, vsp],
        out_specs=row,
        out_shape=jax.ShapeDtypeStruct((N, D), _F32),
    )(nfeat, p, fpad, wn, wa, wf, b1, lg, lb, w2, b2)


def _sc_mesh():
    return plsc.VectorSubcoreMesh(
        core_axis_name="c", subcore_axis_name="s", num_cores=NC, num_subcores=NS)


def _sc_gather_body(gs_hbm, gd_hbm, src_hbm, dst_hbm, out_hbm,
                    idx_s, idx_d, rows,
                    sg0, sg1, sg2, sd0, sd1, sd2, so0, so1, so2):
    sg = (sg0, sg1, sg2)
    sd = (sd0, sd1, sd2)
    so = (so0, so1, so2)
    w = lax.axis_index("s") * NC + lax.axis_index("c")

    def rows_at(b):
        return rows.at[pl.ds(b * CCH_G, CCH_G)]

    def drain(dst_ref, sem):
        pltpu.make_async_copy(gs_hbm.at[pl.ds(0, CCH_G)], dst_ref, sem).wait()

    def slot(j, b):
        j3 = j - 2
        c3 = w + j3 * NW
        b3 = (b - 2) % NB_G

        @pl.when(jnp.logical_and(j3 >= 0, c3 < NCH_G))
        def _():
            drain(rows_at(b3), sd[b3])
            pltpu.async_copy(rows_at(b3), out_hbm.at[pl.ds(c3 * CCH_G, CCH_G)],
                             so[b3])

        j2 = j - 1
        c2 = w + j2 * NW
        b2 = (b - 1) % NB_G

        @pl.when(jnp.logical_and(j2 >= 0, c2 < NCH_G))
        def _():
            drain(rows_at(b2), sg[b2])
            for k in range(KC_G):
                pltpu.async_copy(
                    gd_hbm.at[idx_d.at[pl.ds(b2 * CCH_G + k * KIDX_G, KIDX_G)]],
                    rows.at[pl.ds(b2 * CCH_G + k * KIDX_G, KIDX_G)],
                    sd[b2], add=True)

        c1 = w + j * NW

        @pl.when(c1 < NCH_G)
        def _():
            @pl.when(j >= NB_G)
            def _():
                pltpu.make_async_copy(rows_at(b),
                                      out_hbm.at[pl.ds(0, CCH_G)], so[b]).wait()

            base = c1 * CCH_G
            pltpu.sync_copy(src_hbm.at[pl.ds(base, CCH_G)],
                            idx_s.at[pl.ds(b * CCH_G, CCH_G)])
            pltpu.sync_copy(dst_hbm.at[pl.ds(base, CCH_G)],
                            idx_d.at[pl.ds(b * CCH_G, CCH_G)])
            for k in range(KC_G):
                pltpu.async_copy(
                    gs_hbm.at[idx_s.at[pl.ds(b * CCH_G + k * KIDX_G, KIDX_G)]],
                    rows.at[pl.ds(b * CCH_G + k * KIDX_G, KIDX_G)],
                    sg[b])

    NSLOT = NJ_G + 2
    NGRP = (NSLOT + NB_G - 1) // NB_G

    def group(g, carry):
        for b in range(NB_G):
            slot(g * NB_G + b, b)
        return carry

    lax.fori_loop(0, NGRP, group, 0)
    for b in range(NB_G):
        pltpu.make_async_copy(rows_at(b), out_hbm.at[pl.ds(0, CCH_G)],
                              so[b]).wait()


def _sc_gather(gs, gd, src, dst):
    return pl.kernel(
        _sc_gather_body,
        out_type=jax.ShapeDtypeStruct((EH, D), _F32),
        mesh=_sc_mesh(),
        scratch_types=[
            pltpu.VMEM((NB_G * CCH_G,), jnp.int32),
            pltpu.VMEM((NB_G * CCH_G,), jnp.int32),
            pltpu.VMEM((NB_G * CCH_G, D), _F32),
        ] + [pltpu.SemaphoreType.DMA] * 9,
    )(gs, gd, src, dst)


def _sc_scatter_body(e_hbm, dst_hbm, out_hbm, idx2, rows, acc,
                     se0, se1, sa0, sa1):
    se = (se0, se1)
    sa = (sa0, sa1)
    cid = lax.axis_index("c")
    sid = lax.axis_index("s")
    w = sid * NC + cid
    zero = jnp.zeros((16,), _F32)

    def zrow(i, carry):
        for j in range(D // 16):
            rows[i, pl.ds(j * 16, 16)] = zero
        return carry

    lax.fori_loop(0, NB_S * CCH_S, zrow, 0)
    z0 = sid * SLAB
    pltpu.sync_copy(rows, acc.at[pl.ds(z0, NB_S * CCH_S)])
    pltpu.sync_copy(rows, acc.at[pl.ds(z0 + NB_S * CCH_S, NB_S * CCH_S)])
    plsc.subcore_barrier()

    def rows_at(b):
        return rows.at[pl.ds(b * CCH_S, CCH_S)]

    def slot(j, b):
        j2 = j - 1
        c2 = w + j2 * NW
        b2 = (b - 1) % NB_S

        @pl.when(jnp.logical_and(j2 >= 0, c2 < NCH_S))
        def _():
            pltpu.make_async_copy(e_hbm.at[pl.ds(0, CCH_S)], rows_at(b2),
                                  se[b2]).wait()
            for k in range(KC_S):
                pltpu.async_copy(
                    rows.at[pl.ds(b2 * CCH_S + k * KIDX_S, KIDX_S)],
                    acc.at[idx2.at[b2 * KC_S + k]], sa[b2], add=True)

        c1 = w + j * NW

        @pl.when(c1 < NCH_S)
        def _():
            @pl.when(j >= NB_S)
            def _():
                for k in range(KC_S):
                    pltpu.make_async_copy(
                        rows.at[pl.ds(b * CCH_S + k * KIDX_S, KIDX_S)],
                        acc.at[pl.ds(0, KIDX_S)], sa[b]).wait()

            base = c1 * CCH_S
            for k in range(KC_S):
                pltpu.sync_copy(dst_hbm.at[pl.ds(base + k * KIDX_S, KIDX_S)],
                                idx2.at[b * KC_S + k])
            pltpu.async_copy(e_hbm.at[pl.ds(base, CCH_S)], rows_at(b), se[b])

    NSLOT = NJ_S + 1
    NGRP = (NSLOT + NB_S - 1) // NB_S

    def group(g, carry):
        for b in range(NB_S):
            slot(g * NB_S + b, b)
        return carry

    lax.fori_loop(0, NGRP, group, 0)
    for b in range(NB_S):
        for k in range(KC_S):
            pltpu.make_async_copy(
                rows.at[pl.ds(b * CCH_S + k * KIDX_S, KIDX_S)],
                acc.at[pl.ds(0, KIDX_S)], sa[b]).wait()
    plsc.subcore_barrier()
    pltpu.sync_copy(acc.at[pl.ds(sid * SLAB, SLAB)],
                    out_hbm.at[pl.ds(cid * NPAD + sid * SLAB, SLAB)])


def _sc_scatter(efeat, dst):
    return pl.kernel(
        _sc_scatter_body,
        out_type=jax.ShapeDtypeStruct((2 * NPAD, D), _F32),
        mesh=_sc_mesh(),
        scratch_types=[
            pltpu.VMEM((NB_S * KC_S, KIDX_S), jnp.int32),
            pltpu.VMEM((NB_S * CCH_S, D), _F32),
            pltpu.VMEM_SHARED((NPAD, D), _F32),
        ] + [pltpu.SemaphoreType.DMA] * 4,
    )(efeat, dst)


def kernel(node_features, edge_features, flow_features, edge_index, params):
    srcs = [edge_index[0, h * EH:(h + 1) * EH] for h in range(NSPLIT)]
    dsts = [edge_index[1, h * EH:(h + 1) * EH] for h in range(NSPLIT)]
    fpad = jnp.pad(flow_features, ((0, 0), (0, D - DF)))
    nfeat = node_features
    efeat = [edge_features[h * EH:(h + 1) * EH] for h in range(NSPLIT)]
    for i in range(P):
        ep = params["edge"][i]
        w1 = ep["w1"]
        w1e = w1[0:D]
        w1s = w1[D:2 * D]
        w1d = w1[2 * D:3 * D]
        w1fs = jnp.pad(w1[3 * D:3 * D + DF], ((0, D - DF), (0, 0)))
        w1fd = jnp.pad(w1[3 * D + DF:], ((0, D - DF), (0, 0)))
        gs, gd = _tables(nfeat, fpad, w1s, w1d, w1fs, w1fd,
                         ep["b1"].reshape(1, D))
        xs = [_sc_gather(gs, gd, srcs[h], dsts[h]) for h in range(NSPLIT)]
        efeat = [_edge_mlp(efeat[h], xs[h], w1e, ep["w2"],
                           ep["g"].reshape(1, D), ep["be"].reshape(1, D),
                           ep["b2"].reshape(1, D)) for h in range(NSPLIT)]
        ps = [_sc_scatter(efeat[h], dsts[h]).reshape(2, NPAD, D)
              for h in range(NSPLIT)]
        p = jnp.concatenate(ps, axis=0)

        np_ = params["node"][i]
        nw1 = np_["w1"]
        wn = nw1[0:D]
        wa = nw1[D:2 * D]
        wf = jnp.pad(nw1[2 * D:], ((0, D - DF), (0, 0)))
        nfeat = _node_mlp(nfeat, p, fpad, wn, wa, wf,
                          np_["b1"].reshape(1, D), np_["g"].reshape(1, D),
                          np_["be"].reshape(1, D), np_["w2"],
                          np_["b2"].reshape(1, D))
    return nfeat
